# baseline (speedup 1.0000x reference)
import functools

import jax
import jax.numpy as jnp
from jax import lax
from jax.experimental import pallas as pl
from jax.experimental.pallas import tpu as pltpu
from jax.experimental.pallas import tpu_sc as plsc

_LANES = 16


@functools.cache
def _build_tw(V, D):
    BLK = 262144
    grid = (V + BLK - 1) // BLK

    def tw_body(w_ref, tt_ref, out_ref):
        out_ref[...] = jnp.sum(tt_ref[...] * w_ref[...].reshape(D, 1), axis=0)

    return pl.pallas_call(
        tw_body,
        grid=(grid,),
        in_specs=[
            pl.BlockSpec((1, D), lambda i: (0, 0)),
            pl.BlockSpec((D, BLK), lambda i: (0, i)),
        ],
        out_specs=pl.BlockSpec((BLK,), lambda i: (i,)),
        out_shape=jax.ShapeDtypeStruct((V,), jnp.float32),
    )


@functools.cache
def _build_pool(B, V, L):
    NW = 32
    S = B // NW
    HALF = L // 2
    NHF = HALF // _LANES
    REM = HALF - NHF * _LANES
    STG = 62496

    mesh = plsc.VectorSubcoreMesh(core_axis_name="c", subcore_axis_name="s")

    @functools.partial(
        pl.kernel,
        mesh=mesh,
        compiler_params=pltpu.CompilerParams(
            needs_layout_passes=False, use_tc_tiling_on_sc=False),
        out_type=jax.ShapeDtypeStruct((B,), jnp.float32),
        scratch_types=[
            pltpu.VMEM_SHARED((V,), jnp.float32),
            pltpu.VMEM((S, 2, HALF), jnp.int32),
            pltpu.VMEM((2, 2, HALF), jnp.float32),
            pltpu.VMEM((S,), jnp.float32),
            pltpu.VMEM((_LANES,), jnp.float32),
            pltpu.VMEM((_LANES,), jnp.float32),
            pltpu.SemaphoreType.DMA,
            pltpu.SemaphoreType.DMA,
            pltpu.SemaphoreType.DMA,
        ],
    )
    def pooled(src_hbm, tw_hbm, bv_hbm, out_hbm,
               tw_sh, idx_v, vals_v, out_v, tw0_v, bv_v,
               gsem0, gsem1, isem):
        cid = lax.axis_index("c")
        sid = lax.axis_index("s")
        wid = sid * 2 + cid
        base = wid * S
        gsems = (gsem0, gsem1)

        icp = pltpu.make_async_copy(src_hbm.at[pl.ds(base, S)], idx_v, isem)
        icp.start()
        pltpu.sync_copy(bv_hbm, bv_v)

        off = sid * STG
        pltpu.sync_copy(tw_hbm.at[pl.ds(off, STG)], tw_sh.at[pl.ds(off, STG)])

        @pl.when(sid == 0)
        def _():
            tail = STG * _LANES
            pltpu.sync_copy(tw_hbm.at[pl.ds(tail, V - STG * _LANES)],
                            tw_sh.at[pl.ds(tail, V - STG * _LANES)])

        plsc.subcore_barrier()

        lanes = lax.iota(jnp.int32, _LANES)
        lane0 = lanes == 0
        zero = jnp.zeros((_LANES,), jnp.float32)
        one = jnp.ones((_LANES,), jnp.float32)

        def allsum(x):
            for sft in (8, 4, 2, 1):
                x = x + jnp.take_along_axis(x, lanes ^ sft, axis=0)
            return x

        pltpu.sync_copy(tw_sh.at[pl.ds(0, _LANES)], tw0_v)
        tw0 = tw0_v[...][0]
        bs = bv_v[...][0]
        l_f = jnp.float32(L)

        icp.wait()

        def g_copy(s, buf, h):
            return pltpu.make_async_copy(
                tw_sh.at[idx_v.at[s, h]],
                vals_v.at[buf, h],
                gsems[buf],
            )

        def g_fire(s, buf):
            for h in range(2):
                g_copy(s, buf, h).start()

        def g_wait(s, buf):
            for h in range(2):
                g_copy(s, buf, h).wait()

        def process(s, buf):
            vsum = zero
            macc = zero
            for h in range(2):
                for k in range(NHF):
                    vsum = vsum + vals_v[buf, h, pl.ds(k * _LANES, _LANES)]
                    chunk = idx_v[s, h, pl.ds(k * _LANES, _LANES)]
                    macc = macc + jnp.where(chunk != 0, one, zero)
                if REM:
                    tailm = lanes >= _LANES - REM
                    tail = vals_v[buf, h, pl.ds(HALF - _LANES, _LANES)]
                    vsum = vsum + jnp.where(tailm, tail, zero)
                    chunk = idx_v[s, h, pl.ds(HALF - _LANES, _LANES)]
                    new = jnp.logical_and(chunk != 0, tailm)
                    macc = macc + jnp.where(new, one, zero)
            len_v = allsum(macc)
            tot = allsum(vsum)
            logit_v = (tot - (l_f - len_v) * tw0) / len_v + bs
            plsc.store_scatter(
                out_v,
                [jnp.broadcast_to(s, (_LANES,)).astype(jnp.int32)],
                logit_v,
                mask=lane0,
            )

        g_fire(0, 0)

        def seq_body(g, carry):
            s0 = 2 * g
            s1 = s0 + 1
            g_fire(s1, 1)
            g_wait(s0, 0)
            process(s0, 0)
            nxt = lax.rem(s0 + 2, S)
            g_fire(nxt, 0)
            g_wait(s1, 1)
            process(s1, 1)
            return carry

        lax.fori_loop(0, S // 2, seq_body, 0)
        g_wait(0, 0)

        for g in range(S // _LANES):
            v = out_v[pl.ds(g * _LANES, _LANES)]
            out_v[pl.ds(g * _LANES, _LANES)] = 1.0 / (1.0 + jnp.exp(-v))

        pltpu.sync_copy(out_v, out_hbm.at[pl.ds(base, S)])

    return pooled


def kernel(src, table, W, b):
    B, L = src.shape
    V, D = table.shape
    src_p = src.reshape(B, 2, L // 2)
    tw = _build_tw(V, D)(W.astype(jnp.float32), table.T)
    bv = jnp.concatenate([
        b.reshape(-1).astype(jnp.float32),
        jnp.zeros((_LANES - 1,), jnp.float32),
    ])
    out = _build_pool(B, V, L)(src_p, tw, bv)
    return out.reshape(B, 1)


# speedup vs baseline: 1.0010x; 1.0010x over previous
import functools

import jax
import jax.numpy as jnp
from jax import lax
from jax.experimental import pallas as pl
from jax.experimental.pallas import tpu as pltpu
from jax.experimental.pallas import tpu_sc as plsc

_LANES = 16


@functools.cache
def _build_tw(V, D):
    BLK = 131072
    grid = (V + BLK - 1) // BLK

    def tw_body(w_ref, tt_ref, out_ref):
        out_ref[...] = jnp.sum(tt_ref[...] * w_ref[...].reshape(D, 1), axis=0)

    return pl.pallas_call(
        tw_body,
        grid=(grid,),
        in_specs=[
            pl.BlockSpec((1, D), lambda i: (0, 0)),
            pl.BlockSpec((D, BLK), lambda i: (0, i)),
        ],
        out_specs=pl.BlockSpec((BLK,), lambda i: (i,)),
        out_shape=jax.ShapeDtypeStruct((V,), jnp.float32),
    )


@functools.cache
def _build_pool(B, V, L):
    NW = 32
    S = B // NW
    HALF = L // 2
    NHF = HALF // _LANES
    REM = HALF - NHF * _LANES
    STG = 62496

    mesh = plsc.VectorSubcoreMesh(core_axis_name="c", subcore_axis_name="s")

    @functools.partial(
        pl.kernel,
        mesh=mesh,
        compiler_params=pltpu.CompilerParams(
            needs_layout_passes=False, use_tc_tiling_on_sc=False),
        out_type=jax.ShapeDtypeStruct((B,), jnp.float32),
        scratch_types=[
            pltpu.VMEM_SHARED((V,), jnp.float32),
            pltpu.VMEM((S, 2, HALF), jnp.int32),
            pltpu.VMEM((2, 2, HALF), jnp.float32),
            pltpu.VMEM((S,), jnp.float32),
            pltpu.VMEM((_LANES,), jnp.float32),
            pltpu.VMEM((_LANES,), jnp.float32),
            pltpu.SemaphoreType.DMA,
            pltpu.SemaphoreType.DMA,
            pltpu.SemaphoreType.DMA,
        ],
    )
    def pooled(src_hbm, tw_hbm, bv_hbm, out_hbm,
               tw_sh, idx_v, vals_v, out_v, tw0_v, bv_v,
               gsem0, gsem1, isem):
        cid = lax.axis_index("c")
        sid = lax.axis_index("s")
        wid = sid * 2 + cid
        base = wid * S
        gsems = (gsem0, gsem1)

        icp = pltpu.make_async_copy(src_hbm.at[pl.ds(base, S)], idx_v, isem)
        icp.start()
        pltpu.sync_copy(bv_hbm, bv_v)

        off = sid * STG
        pltpu.sync_copy(tw_hbm.at[pl.ds(off, STG)], tw_sh.at[pl.ds(off, STG)])

        @pl.when(sid == 0)
        def _():
            tail = STG * _LANES
            pltpu.sync_copy(tw_hbm.at[pl.ds(tail, V - STG * _LANES)],
                            tw_sh.at[pl.ds(tail, V - STG * _LANES)])

        plsc.subcore_barrier()

        lanes = lax.iota(jnp.int32, _LANES)
        lane0 = lanes == 0
        zero = jnp.zeros((_LANES,), jnp.float32)
        one = jnp.ones((_LANES,), jnp.float32)

        def allsum(x):
            for sft in (8, 4, 2, 1):
                x = x + jnp.take_along_axis(x, lanes ^ sft, axis=0)
            return x

        pltpu.sync_copy(tw_sh.at[pl.ds(0, _LANES)], tw0_v)
        tw0 = tw0_v[...][0]
        bs = bv_v[...][0]
        l_f = jnp.float32(L)

        icp.wait()

        def g_copy(s, buf, h):
            return pltpu.make_async_copy(
                tw_sh.at[idx_v.at[s, h]],
                vals_v.at[buf, h],
                gsems[buf],
            )

        def g_fire(s, buf):
            for h in range(2):
                g_copy(s, buf, h).start()

        def g_wait(s, buf):
            for h in range(2):
                g_copy(s, buf, h).wait()

        def process(s, buf):
            vsum = zero
            macc = zero
            for h in range(2):
                for k in range(NHF):
                    vsum = vsum + vals_v[buf, h, pl.ds(k * _LANES, _LANES)]
                    chunk = idx_v[s, h, pl.ds(k * _LANES, _LANES)]
                    macc = macc + jnp.where(chunk != 0, one, zero)
                if REM:
                    tailm = lanes >= _LANES - REM
                    tail = vals_v[buf, h, pl.ds(HALF - _LANES, _LANES)]
                    vsum = vsum + jnp.where(tailm, tail, zero)
                    chunk = idx_v[s, h, pl.ds(HALF - _LANES, _LANES)]
                    new = jnp.logical_and(chunk != 0, tailm)
                    macc = macc + jnp.where(new, one, zero)
            len_v = allsum(macc)
            tot = allsum(vsum)
            logit_v = (tot - (l_f - len_v) * tw0) / len_v + bs
            plsc.store_scatter(
                out_v,
                [jnp.broadcast_to(s, (_LANES,)).astype(jnp.int32)],
                logit_v,
                mask=lane0,
            )

        g_fire(0, 0)

        def seq_body(g, carry):
            s0 = 2 * g
            s1 = s0 + 1
            g_fire(s1, 1)
            g_wait(s0, 0)
            process(s0, 0)
            nxt = lax.rem(s0 + 2, S)
            g_fire(nxt, 0)
            g_wait(s1, 1)
            process(s1, 1)
            return carry

        lax.fori_loop(0, S // 2, seq_body, 0)
        g_wait(0, 0)

        for g in range(S // _LANES):
            v = out_v[pl.ds(g * _LANES, _LANES)]
            out_v[pl.ds(g * _LANES, _LANES)] = 1.0 / (1.0 + jnp.exp(-v))

        pltpu.sync_copy(out_v, out_hbm.at[pl.ds(base, S)])

    return pooled


def kernel(src, table, W, b):
    B, L = src.shape
    V, D = table.shape
    src_p = src.reshape(B, 2, L // 2)
    tw = _build_tw(V, D)(W.astype(jnp.float32), table.T)
    bv = jnp.concatenate([
        b.reshape(-1).astype(jnp.float32),
        jnp.zeros((_LANES - 1,), jnp.float32),
    ])
    out = _build_pool(B, V, L)(src_p, tw, bv)
    return out.reshape(B, 1)
